# baseline (speedup 1.0000x reference)
import functools

import jax
import jax.numpy as jnp
from jax.experimental import pallas as pl
from jax.experimental.pallas import tpu as pltpu

H = 3
K = 87
TAU = 1.0
D = 768


def _fused_kernel(x_ref, mask_ref, k_ref, f_ref, s_ref, *, bb, n):
    prec = jax.lax.Precision.DEFAULT
    for b in range(bb):
        xb = x_ref[b]
        xn2 = jnp.sum(xb * xb, axis=1, keepdims=True)
        invs = []
        for h in range(H):
            kh = k_ref[h]
            kn2 = jnp.sum(kh * kh, axis=1)[None, :]
            g = jax.lax.dot_general(
                xb, kh, (((1,), (1,)), ((), ())),
                precision=prec, preferred_element_type=jnp.float32)
            d2 = jnp.maximum(xn2 + kn2 - 2.0 * g, 0.0)
            m = jnp.mean(d2, axis=1, keepdims=True)
            invs.append(1.0 / ((d2 / m) / TAU))

        s_pre = jnp.maximum(jnp.maximum(invs[0], invs[1]), invs[2])
        s_pre = s_pre * mask_ref[b] * 5.0
        z = s_pre - jnp.max(s_pre, axis=1, keepdims=True)
        e = jnp.exp(z)
        s = e / jnp.sum(e, axis=1, keepdims=True)
        s_ref[b] = s

        a = [jnp.max(invs[h], axis=0, keepdims=True) for h in range(H)]
        pick2 = a[2] > jnp.maximum(a[0], a[1])
        pick1 = jnp.logical_and(a[1] > a[0], jnp.logical_not(pick2))
        pick0 = jnp.logical_not(jnp.logical_or(pick1, pick2))

        acc = jnp.zeros((n, D), dtype=jnp.float32)
        for h, p in enumerate((pick0, pick1, pick2)):
            th = s * p.astype(jnp.float32)
            acc = acc + jax.lax.dot_general(
                th, k_ref[h], (((1,), (0,)), ((), ())),
                precision=prec, preferred_element_type=jnp.float32)
        f_ref[b] = acc


def kernel(x, mask, k_param, W_lin):
    B, N, _ = x.shape
    bb = 8
    grid = (B // bb,)
    mask_f = mask.astype(jnp.float32)[:, :, None]

    f, s = pl.pallas_call(
        functools.partial(_fused_kernel, bb=bb, n=N),
        grid=grid,
        in_specs=[
            pl.BlockSpec((bb, N, D), lambda i: (i, 0, 0)),
            pl.BlockSpec((bb, N, 1), lambda i: (i, 0, 0)),
            pl.BlockSpec((H, K, D), lambda i: (0, 0, 0)),
        ],
        out_specs=[
            pl.BlockSpec((bb, N, D), lambda i: (i, 0, 0)),
            pl.BlockSpec((bb, N, K), lambda i: (i, 0, 0)),
        ],
        out_shape=[
            jax.ShapeDtypeStruct((B, N, D), jnp.float32),
            jax.ShapeDtypeStruct((B, N, K), jnp.float32),
        ],
        compiler_params=pltpu.CompilerParams(
            dimension_semantics=("parallel",),
        ),
    )(x, mask_f, k_param)

    k_out = jnp.transpose(k_param, (1, 0, 2)).reshape(-1, D)
    return f, s, k_out


# speedup vs baseline: 1.8139x; 1.8139x over previous
import functools

import jax
import jax.numpy as jnp
from jax.experimental import pallas as pl
from jax.experimental.pallas import tpu as pltpu

H = 3
K = 87
TAU = 1.0
D = 768


def _fused_kernel(x_ref, mask_ref, k_ref, f_ref, s_ref, *, bb, n):
    rows = bb * n
    prec = jax.lax.Precision.DEFAULT
    x2 = jnp.concatenate([x_ref[b] for b in range(bb)], axis=0)
    maskc = jnp.concatenate([mask_ref[b] for b in range(bb)], axis=0)
    xn2 = jnp.sum(x2 * x2, axis=1, keepdims=True)

    invs = []
    for h in range(H):
        kh = k_ref[h]
        kn2 = jnp.sum(kh * kh, axis=1)[None, :]
        g = jax.lax.dot_general(
            x2, kh, (((1,), (1,)), ((), ())),
            precision=prec, preferred_element_type=jnp.float32)
        d2 = jnp.maximum(xn2 + kn2 - 2.0 * g, 0.0)
        m = jnp.mean(d2, axis=1, keepdims=True)
        invs.append(1.0 / ((d2 / m) / TAU))

    s_pre = jnp.maximum(jnp.maximum(invs[0], invs[1]), invs[2])
    s_pre = s_pre * maskc * 5.0
    z = s_pre - jnp.max(s_pre, axis=1, keepdims=True)
    e = jnp.exp(z)
    s = e / jnp.sum(e, axis=1, keepdims=True)

    a = []
    for h in range(H):
        a.append(jnp.concatenate(
            [jnp.max(invs[h][b * n:(b + 1) * n, :], axis=0, keepdims=True)
             for b in range(bb)], axis=0))
    pick2 = a[2] > jnp.maximum(a[0], a[1])
    pick1 = jnp.logical_and(a[1] > a[0], jnp.logical_not(pick2))
    pick0 = jnp.logical_not(jnp.logical_or(pick1, pick2))
    picks = (pick0, pick1, pick2)

    grp = jax.lax.broadcasted_iota(jnp.int32, (rows, bb), 0) // n
    lane = jax.lax.broadcasted_iota(jnp.int32, (rows, bb), 1)
    expand = (grp == lane).astype(jnp.float32)

    acc = jnp.zeros((rows, D), dtype=jnp.float32)
    for h in range(H):
        sel = jax.lax.dot_general(
            expand, picks[h].astype(jnp.float32), (((1,), (0,)), ((), ())),
            precision=prec, preferred_element_type=jnp.float32)
        acc = acc + jax.lax.dot_general(
            s * sel, k_ref[h], (((1,), (0,)), ((), ())),
            precision=prec, preferred_element_type=jnp.float32)

    for b in range(bb):
        f_ref[b] = acc[b * n:(b + 1) * n, :]
        s_ref[b] = s[b * n:(b + 1) * n, :]


def kernel(x, mask, k_param, W_lin):
    B, N, _ = x.shape
    bb = 8
    grid = (B // bb,)
    mask_f = mask.astype(jnp.float32)[:, :, None]

    f, s = pl.pallas_call(
        functools.partial(_fused_kernel, bb=bb, n=N),
        grid=grid,
        in_specs=[
            pl.BlockSpec((bb, N, D), lambda i: (i, 0, 0)),
            pl.BlockSpec((bb, N, 1), lambda i: (i, 0, 0)),
            pl.BlockSpec((H, K, D), lambda i: (0, 0, 0)),
        ],
        out_specs=[
            pl.BlockSpec((bb, N, D), lambda i: (i, 0, 0)),
            pl.BlockSpec((bb, N, K), lambda i: (i, 0, 0)),
        ],
        out_shape=[
            jax.ShapeDtypeStruct((B, N, D), jnp.float32),
            jax.ShapeDtypeStruct((B, N, K), jnp.float32),
        ],
        compiler_params=pltpu.CompilerParams(
            dimension_semantics=("parallel",),
        ),
    )(x, mask_f, k_param)

    k_out = jnp.transpose(k_param, (1, 0, 2)).reshape(-1, D)
    return f, s, k_out


# speedup vs baseline: 1.9610x; 1.0811x over previous
import functools

import jax
import jax.numpy as jnp
from jax.experimental import pallas as pl
from jax.experimental.pallas import tpu as pltpu

H = 3
K = 87
TAU = 1.0
D = 768


def _fused_kernel(x_ref, mask_ref, k_ref, f_ref, s_ref, *, bb, n):
    rows = bb * n
    prec = jax.lax.Precision.DEFAULT
    x2 = jnp.concatenate([x_ref[b] for b in range(bb)], axis=0)
    maskc = jnp.concatenate([mask_ref[b] for b in range(bb)], axis=0)
    xn2 = jnp.sum(x2 * x2, axis=1, keepdims=True)

    invs = []
    for h in range(H):
        kh = k_ref[h]
        kn2 = jnp.sum(kh * kh, axis=1)[None, :]
        g = jax.lax.dot_general(
            x2, kh, (((1,), (1,)), ((), ())),
            precision=prec, preferred_element_type=jnp.float32)
        d2 = jnp.maximum(xn2 + kn2 - 2.0 * g, 0.0)
        m = jnp.mean(d2, axis=1, keepdims=True)
        invs.append(1.0 / ((d2 / m) / TAU))

    s_pre = jnp.maximum(jnp.maximum(invs[0], invs[1]), invs[2])
    s_pre = s_pre * maskc * 5.0
    z = s_pre - jnp.max(s_pre, axis=1, keepdims=True)
    e = jnp.exp(z)
    s = e / jnp.sum(e, axis=1, keepdims=True)

    a = []
    for h in range(H):
        a.append(jnp.concatenate(
            [jnp.max(invs[h][b * n:(b + 1) * n, :], axis=0, keepdims=True)
             for b in range(bb)], axis=0))
    pick2 = a[2] > jnp.maximum(a[0], a[1])
    pick1 = jnp.logical_and(a[1] > a[0], jnp.logical_not(pick2))
    pick0 = jnp.logical_not(jnp.logical_or(pick1, pick2))
    picks = (pick0, pick1, pick2)

    grp = jax.lax.broadcasted_iota(jnp.int32, (rows, bb), 0) // n
    lane = jax.lax.broadcasted_iota(jnp.int32, (rows, bb), 1)
    expand = (grp == lane).astype(jnp.float32)

    acc = jnp.zeros((rows, D), dtype=jnp.float32)
    for h in range(H):
        sel = jax.lax.dot_general(
            expand, picks[h].astype(jnp.float32), (((1,), (0,)), ((), ())),
            precision=prec, preferred_element_type=jnp.float32)
        acc = acc + jax.lax.dot_general(
            s * sel, k_ref[h], (((1,), (0,)), ((), ())),
            precision=prec, preferred_element_type=jnp.float32)

    for b in range(bb):
        f_ref[b] = acc[b * n:(b + 1) * n, :]
        s_ref[b] = s[b * n:(b + 1) * n, :]


def kernel(x, mask, k_param, W_lin):
    B, N, _ = x.shape
    bb = 16
    grid = (B // bb,)
    mask_f = mask.astype(jnp.float32)[:, :, None]

    f, s = pl.pallas_call(
        functools.partial(_fused_kernel, bb=bb, n=N),
        grid=grid,
        in_specs=[
            pl.BlockSpec((bb, N, D), lambda i: (i, 0, 0)),
            pl.BlockSpec((bb, N, 1), lambda i: (i, 0, 0)),
            pl.BlockSpec((H, K, D), lambda i: (0, 0, 0)),
        ],
        out_specs=[
            pl.BlockSpec((bb, N, D), lambda i: (i, 0, 0)),
            pl.BlockSpec((bb, N, K), lambda i: (i, 0, 0)),
        ],
        out_shape=[
            jax.ShapeDtypeStruct((B, N, D), jnp.float32),
            jax.ShapeDtypeStruct((B, N, K), jnp.float32),
        ],
        compiler_params=pltpu.CompilerParams(
            dimension_semantics=("parallel",),
        ),
    )(x, mask_f, k_param)

    k_out = jnp.transpose(k_param, (1, 0, 2)).reshape(-1, D)
    return f, s, k_out


# speedup vs baseline: 2.0097x; 1.0248x over previous
import functools

import jax
import jax.numpy as jnp
from jax.experimental import pallas as pl
from jax.experimental.pallas import tpu as pltpu

H = 3
K = 87
TAU = 1.0
D = 768


def _fused_kernel(x_ref, mask_ref, k_ref, f_ref, s_ref, *, bb, n):
    rows = bb * n
    prec = jax.lax.Precision.DEFAULT
    x2 = jnp.concatenate([x_ref[b] for b in range(bb)], axis=0)
    maskc = jnp.concatenate([mask_ref[b] for b in range(bb)], axis=0)
    xn2 = jnp.sum(x2 * x2, axis=1, keepdims=True)

    invs = []
    for h in range(H):
        kh = k_ref[h]
        kn2 = jnp.sum(kh * kh, axis=1)[None, :]
        g = jax.lax.dot_general(
            x2, kh, (((1,), (1,)), ((), ())),
            precision=prec, preferred_element_type=jnp.float32)
        d2 = jnp.maximum(xn2 + kn2 - 2.0 * g, 0.0)
        m = jnp.mean(d2, axis=1, keepdims=True)
        invs.append(1.0 / ((d2 / m) / TAU))

    s_pre = jnp.maximum(jnp.maximum(invs[0], invs[1]), invs[2])
    s_pre = s_pre * maskc * 5.0
    z = s_pre - jnp.max(s_pre, axis=1, keepdims=True)
    e = jnp.exp(z)
    s = e / jnp.sum(e, axis=1, keepdims=True)

    a = []
    for h in range(H):
        a.append(jnp.concatenate(
            [jnp.max(invs[h][b * n:(b + 1) * n, :], axis=0, keepdims=True)
             for b in range(bb)], axis=0))
    pick2 = a[2] > jnp.maximum(a[0], a[1])
    pick1 = jnp.logical_and(a[1] > a[0], jnp.logical_not(pick2))
    pick0 = jnp.logical_not(jnp.logical_or(pick1, pick2))
    picks = (pick0, pick1, pick2)

    grp = jax.lax.broadcasted_iota(jnp.int32, (rows, bb), 0) // n
    lane = jax.lax.broadcasted_iota(jnp.int32, (rows, bb), 1)
    expand = (grp == lane).astype(jnp.float32)

    acc = jnp.zeros((rows, D), dtype=jnp.float32)
    for h in range(H):
        sel = jax.lax.dot_general(
            expand, picks[h].astype(jnp.float32), (((1,), (0,)), ((), ())),
            precision=prec, preferred_element_type=jnp.float32)
        acc = acc + jax.lax.dot_general(
            s * sel, k_ref[h], (((1,), (0,)), ((), ())),
            precision=prec, preferred_element_type=jnp.float32)

    for b in range(bb):
        f_ref[b] = acc[b * n:(b + 1) * n, :]
        s_ref[b] = s[b * n:(b + 1) * n, :]


def kernel(x, mask, k_param, W_lin):
    B, N, _ = x.shape
    bb = 32
    grid = (B // bb,)
    mask_f = mask.astype(jnp.float32)[:, :, None]

    f, s = pl.pallas_call(
        functools.partial(_fused_kernel, bb=bb, n=N),
        grid=grid,
        in_specs=[
            pl.BlockSpec((bb, N, D), lambda i: (i, 0, 0)),
            pl.BlockSpec((bb, N, 1), lambda i: (i, 0, 0)),
            pl.BlockSpec((H, K, D), lambda i: (0, 0, 0)),
        ],
        out_specs=[
            pl.BlockSpec((bb, N, D), lambda i: (i, 0, 0)),
            pl.BlockSpec((bb, N, K), lambda i: (i, 0, 0)),
        ],
        out_shape=[
            jax.ShapeDtypeStruct((B, N, D), jnp.float32),
            jax.ShapeDtypeStruct((B, N, K), jnp.float32),
        ],
        compiler_params=pltpu.CompilerParams(
            dimension_semantics=("parallel",),
        ),
    )(x, mask_f, k_param)

    k_out = jnp.transpose(k_param, (1, 0, 2)).reshape(-1, D)
    return f, s, k_out


# speedup vs baseline: 2.0341x; 1.0121x over previous
import functools

import jax
import jax.numpy as jnp
from jax.experimental import pallas as pl
from jax.experimental.pallas import tpu as pltpu

H = 3
K = 87
TAU = 1.0
D = 768


def _fused_kernel(x_ref, mask_ref, k_ref, f_ref, s_ref, *, bb, n):
    rows = bb * n
    prec = jax.lax.Precision.DEFAULT
    x2 = jnp.concatenate([x_ref[b] for b in range(bb)], axis=0)
    maskc = jnp.concatenate([mask_ref[b] for b in range(bb)], axis=0)
    xn2 = jnp.sum(x2 * x2, axis=1, keepdims=True)

    invs = []
    for h in range(H):
        kh = k_ref[h]
        kn2 = jnp.sum(kh * kh, axis=1)[None, :]
        g = jax.lax.dot_general(
            x2, kh, (((1,), (1,)), ((), ())),
            precision=prec, preferred_element_type=jnp.float32)
        d2 = jnp.maximum(xn2 + kn2 - 2.0 * g, 0.0)
        m = jnp.mean(d2, axis=1, keepdims=True)
        invs.append(1.0 / ((d2 / m) / TAU))

    s_pre = jnp.maximum(jnp.maximum(invs[0], invs[1]), invs[2])
    s_pre = s_pre * maskc * 5.0
    z = s_pre - jnp.max(s_pre, axis=1, keepdims=True)
    e = jnp.exp(z)
    s = e / jnp.sum(e, axis=1, keepdims=True)

    a = []
    for h in range(H):
        a.append(jnp.concatenate(
            [jnp.max(invs[h][b * n:(b + 1) * n, :], axis=0, keepdims=True)
             for b in range(bb)], axis=0))
    pick2 = a[2] > jnp.maximum(a[0], a[1])
    pick1 = jnp.logical_and(a[1] > a[0], jnp.logical_not(pick2))
    pick0 = jnp.logical_not(jnp.logical_or(pick1, pick2))
    picks = (pick0, pick1, pick2)

    grp = jax.lax.broadcasted_iota(jnp.int32, (rows, bb), 0) // n
    lane = jax.lax.broadcasted_iota(jnp.int32, (rows, bb), 1)
    expand = (grp == lane).astype(jnp.float32)

    acc = jnp.zeros((rows, D), dtype=jnp.float32)
    for h in range(H):
        sel = jax.lax.dot_general(
            expand, picks[h].astype(jnp.float32), (((1,), (0,)), ((), ())),
            precision=prec, preferred_element_type=jnp.float32)
        acc = acc + jax.lax.dot_general(
            s * sel, k_ref[h], (((1,), (0,)), ((), ())),
            precision=prec, preferred_element_type=jnp.float32)

    for b in range(bb):
        f_ref[b] = acc[b * n:(b + 1) * n, :]
        s_ref[b] = s[b * n:(b + 1) * n, :]


def kernel(x, mask, k_param, W_lin):
    B, N, _ = x.shape
    bb = 64
    grid = (B // bb,)
    mask_f = mask.astype(jnp.float32)[:, :, None]

    f, s = pl.pallas_call(
        functools.partial(_fused_kernel, bb=bb, n=N),
        grid=grid,
        in_specs=[
            pl.BlockSpec((bb, N, D), lambda i: (i, 0, 0)),
            pl.BlockSpec((bb, N, 1), lambda i: (i, 0, 0)),
            pl.BlockSpec((H, K, D), lambda i: (0, 0, 0)),
        ],
        out_specs=[
            pl.BlockSpec((bb, N, D), lambda i: (i, 0, 0)),
            pl.BlockSpec((bb, N, K), lambda i: (i, 0, 0)),
        ],
        out_shape=[
            jax.ShapeDtypeStruct((B, N, D), jnp.float32),
            jax.ShapeDtypeStruct((B, N, K), jnp.float32),
        ],
        compiler_params=pltpu.CompilerParams(
            dimension_semantics=("parallel",),
        ),
    )(x, mask_f, k_param)

    k_out = jnp.transpose(k_param, (1, 0, 2)).reshape(-1, D)
    return f, s, k_out
